# baseline (speedup 1.0000x reference)
import functools

import jax
import jax.numpy as jnp
from jax import lax
from jax.experimental import pallas as pl
from jax.experimental.pallas import tpu as pltpu
from jax.experimental.pallas import tpu_sc as plsc

K_CODES = 8192
D_LAT = 32
N_TOK = 8192
BETA = 0.25

TB = 1024
KB = 2048
NT = N_TOK // TB
NK = K_CODES // KB

_NC = 2
_NS = 16
_NW = _NC * _NS
_CHUNK = 128
_CHUNKS_TOTAL = N_TOK // _CHUNK
_CHUNKS_PER_W = _CHUNKS_TOTAL // _NW


def _sum32(p):
    r = ((p[:, 0:8] + p[:, 8:16]) + p[:, 16:24]) + p[:, 24:32]
    c = [r[:, j] for j in range(8)]
    return ((c[0] + c[1]) + (c[2] + c[3])) + ((c[4] + c[5]) + (c[6] + c[7]))


def _dist_argmin_body(x_ref, w_ref, idx_ref, loss_ref, minv, mini):
    t = pl.program_id(0)
    k = pl.program_id(1)
    xb = x_ref[...]
    wb = w_ref[...]
    a = _sum32(xb * xb)
    b = _sum32(wb * wb)
    s = lax.dot_general(xb, wb, (((1,), (1,)), ((), ())))
    d2 = ((a[:, None] - 2.0 * s) + b[None, :]) * (1.0 / D_LAT)
    lmin = jnp.min(d2, axis=1)
    kio = lax.broadcasted_iota(jnp.int32, (TB, KB), 1)
    lidx = jnp.min(jnp.where(d2 == lmin[:, None], kio, jnp.int32(2**30)),
                   axis=1) + k * KB

    @pl.when(k == 0)
    def _():
        minv[...] = lmin.astype(jnp.bfloat16)
        mini[...] = lidx

    @pl.when(k > 0)
    def _():
        acc = minv[...].astype(jnp.float32)
        better = lmin < acc
        minv[...] = jnp.where(better, lmin, acc).astype(jnp.bfloat16)
        mini[...] = jnp.where(better, lidx, mini[...])

    @pl.when(k == NK - 1)
    def _():
        idx_ref[0, 0, :] = mini[...]
        tot = jnp.sum(minv[...].astype(jnp.float32))
        acc = jnp.where(t == 0, 0.0, loss_ref[0, 0]) + tot
        loss_ref[0, 0] = acc * jnp.where(t == NT - 1,
                                         (1.0 + BETA) / N_TOK, 1.0)


_dist_argmin = pl.pallas_call(
    _dist_argmin_body,
    grid=(NT, NK),
    in_specs=[
        pl.BlockSpec((TB, D_LAT), lambda t, k: (t, 0)),
        pl.BlockSpec((KB, D_LAT), lambda t, k: (k, 0)),
    ],
    out_specs=[
        pl.BlockSpec((1, 1, TB), lambda t, k: (t, 0, 0)),
        pl.BlockSpec((1, 1), lambda t, k: (0, 0),
                     memory_space=pltpu.SMEM),
    ],
    out_shape=[
        jax.ShapeDtypeStruct((NT, 1, TB), jnp.int32),
        jax.ShapeDtypeStruct((1, 1), jnp.float32),
    ],
    scratch_shapes=[
        pltpu.VMEM((TB,), jnp.bfloat16),
        pltpu.VMEM((TB,), jnp.int32),
    ],
)


@functools.cache
def _make_sc_gather():
    @functools.partial(
        pl.kernel,
        out_type=jax.ShapeDtypeStruct((N_TOK, D_LAT), jnp.float32),
        mesh=plsc.VectorSubcoreMesh(core_axis_name="c", subcore_axis_name="s",
                                    num_cores=_NC, num_subcores=_NS),
        scratch_types=[
            pltpu.VMEM((_CHUNK,), jnp.int32),
            pltpu.VMEM((_CHUNK, D_LAT), jnp.float32),
            pltpu.SemaphoreType.DMA,
        ],
        compiler_params=pltpu.CompilerParams(use_tc_tiling_on_sc=False),
    )
    def _sc_gather(idx_hbm, w_hbm, out_hbm, idx_v, rows_v, sem):
        c = lax.axis_index("c")
        s = lax.axis_index("s")
        wid = s * _NC + c
        for j in range(_CHUNKS_PER_W):
            r = wid * _CHUNKS_PER_W + j
            pltpu.sync_copy(idx_hbm.at[r], idx_v)
            pltpu.async_copy(w_hbm.at[idx_v], rows_v, sem).wait()
            pltpu.sync_copy(rows_v, out_hbm.at[pl.ds(r * _CHUNK, _CHUNK)])

    return _sc_gather


def kernel(x, W):
    xp = jnp.transpose(x, (0, 2, 3, 1))
    flat = xp.reshape(N_TOK, D_LAT)
    idx3, loss_arr = _dist_argmin(flat, W)
    idx = idx3.reshape(N_TOK)
    q = _make_sc_gather()(idx.reshape(_CHUNKS_TOTAL, _CHUNK), W)
    out = jnp.transpose(q.reshape(xp.shape), (0, 3, 1, 2))
    return (loss_arr[0, 0], idx.reshape(N_TOK, 1), out)


# speedup vs baseline: 1.4396x; 1.4396x over previous
import functools

import jax
import jax.numpy as jnp
from jax import lax
from jax.experimental import pallas as pl
from jax.experimental.pallas import tpu as pltpu
from jax.experimental.pallas import tpu_sc as plsc

K_CODES = 8192
D_LAT = 32
N_TOK = 8192
BETA = 0.25

TB = 1024
KB = 2048
NT = N_TOK // TB
NK = K_CODES // KB

_NC = 2
_NS = 16
_NW = _NC * _NS
_CHUNK = 128
_CHUNKS_TOTAL = N_TOK // _CHUNK
_CHUNKS_PER_W = _CHUNKS_TOTAL // _NW


def _sum32(p):
    r = ((p[:, 0:8] + p[:, 8:16]) + p[:, 16:24]) + p[:, 24:32]
    c = [r[:, j] for j in range(8)]
    return ((c[0] + c[1]) + (c[2] + c[3])) + ((c[4] + c[5]) + (c[6] + c[7]))


def _dist_argmin_body(x_ref, w_ref, idx_ref, loss_ref, minv, mini, a_s, b_s):
    t = pl.program_id(0)
    k = pl.program_id(1)
    xb = x_ref[...]
    wb = w_ref[...]

    @pl.when(k == 0)
    def _():
        a_s[...] = _sum32(xb * xb)

    @pl.when(t == 0)
    def _():
        b_s[pl.ds(k * KB, KB)] = _sum32(wb * wb)

    a = a_s[...]
    b = b_s[pl.ds(k * KB, KB)]
    s = lax.dot_general(xb, wb, (((1,), (1,)), ((), ())))
    d2 = ((a[:, None] - 2.0 * s) + b[None, :]) * (1.0 / D_LAT)
    gio = lax.broadcasted_iota(jnp.int32, (TB, 128), 1)
    acc_v = d2[:, 0:128]
    acc_i = gio + k * KB
    for g in range(1, KB // 128):
        dg = d2[:, g * 128:(g + 1) * 128]
        better = dg < acc_v
        acc_v = jnp.where(better, dg, acc_v)
        acc_i = jnp.where(better, gio + (k * KB + g * 128), acc_i)
    lmin = jnp.min(acc_v, axis=1)
    lidx = jnp.min(jnp.where(acc_v == lmin[:, None], acc_i, jnp.int32(2**30)),
                   axis=1)

    @pl.when(k == 0)
    def _():
        minv[...] = lmin.astype(jnp.bfloat16)
        mini[...] = lidx

    @pl.when(k > 0)
    def _():
        acc = minv[...].astype(jnp.float32)
        better = lmin < acc
        minv[...] = jnp.where(better, lmin, acc).astype(jnp.bfloat16)
        mini[...] = jnp.where(better, lidx, mini[...])

    @pl.when(k == NK - 1)
    def _():
        idx_ref[0, 0, :] = mini[...]
        tot = jnp.sum(minv[...].astype(jnp.float32))
        acc = jnp.where(t == 0, 0.0, loss_ref[0, 0]) + tot
        loss_ref[0, 0] = acc * jnp.where(t == NT - 1,
                                         (1.0 + BETA) / N_TOK, 1.0)


_dist_argmin = pl.pallas_call(
    _dist_argmin_body,
    grid=(NT, NK),
    in_specs=[
        pl.BlockSpec((TB, D_LAT), lambda t, k: (t, 0)),
        pl.BlockSpec((KB, D_LAT), lambda t, k: (k, 0)),
    ],
    out_specs=[
        pl.BlockSpec((1, 1, TB), lambda t, k: (t, 0, 0)),
        pl.BlockSpec((1, 1), lambda t, k: (0, 0),
                     memory_space=pltpu.SMEM),
    ],
    out_shape=[
        jax.ShapeDtypeStruct((NT, 1, TB), jnp.int32),
        jax.ShapeDtypeStruct((1, 1), jnp.float32),
    ],
    scratch_shapes=[
        pltpu.VMEM((TB,), jnp.bfloat16),
        pltpu.VMEM((TB,), jnp.int32),
        pltpu.VMEM((TB,), jnp.float32),
        pltpu.VMEM((K_CODES,), jnp.float32),
    ],
)


@functools.cache
def _make_sc_gather():
    @functools.partial(
        pl.kernel,
        out_type=jax.ShapeDtypeStruct((N_TOK, D_LAT), jnp.float32),
        mesh=plsc.VectorSubcoreMesh(core_axis_name="c", subcore_axis_name="s",
                                    num_cores=_NC, num_subcores=_NS),
        scratch_types=[
            pltpu.VMEM((_CHUNK,), jnp.int32),
            pltpu.VMEM((_CHUNK, D_LAT), jnp.float32),
            pltpu.SemaphoreType.DMA,
        ],
        compiler_params=pltpu.CompilerParams(use_tc_tiling_on_sc=False),
    )
    def _sc_gather(idx_hbm, w_hbm, out_hbm, idx_v, rows_v, sem):
        c = lax.axis_index("c")
        s = lax.axis_index("s")
        wid = s * _NC + c
        for j in range(_CHUNKS_PER_W):
            r = wid * _CHUNKS_PER_W + j
            pltpu.sync_copy(idx_hbm.at[r], idx_v)
            pltpu.async_copy(w_hbm.at[idx_v], rows_v, sem).wait()
            pltpu.sync_copy(rows_v, out_hbm.at[pl.ds(r * _CHUNK, _CHUNK)])

    return _sc_gather


def kernel(x, W):
    xp = jnp.transpose(x, (0, 2, 3, 1))
    flat = xp.reshape(N_TOK, D_LAT)
    idx3, loss_arr = _dist_argmin(flat, W)
    idx = idx3.reshape(N_TOK)
    q = _make_sc_gather()(idx.reshape(_CHUNKS_TOTAL, _CHUNK), W)
    out = jnp.transpose(q.reshape(xp.shape), (0, 3, 1, 2))
    return (loss_arr[0, 0], idx.reshape(N_TOK, 1), out)


# speedup vs baseline: 1.6443x; 1.1422x over previous
import functools

import jax
import jax.numpy as jnp
from jax import lax
from jax.experimental import pallas as pl
from jax.experimental.pallas import tpu as pltpu
from jax.experimental.pallas import tpu_sc as plsc

K_CODES = 8192
D_LAT = 32
N_TOK = 8192
BETA = 0.25

TB = 1024
KB = 2048
NT = N_TOK // TB
NK = K_CODES // KB

_NC = 2
_NS = 16
_NW = _NC * _NS
_CHUNK = 128
_CHUNKS_TOTAL = N_TOK // _CHUNK
_CHUNKS_PER_W = _CHUNKS_TOTAL // _NW


def _sum32(p):
    r = ((p[:, 0:8] + p[:, 8:16]) + p[:, 16:24]) + p[:, 24:32]
    c = [r[:, j] for j in range(8)]
    return ((c[0] + c[1]) + (c[2] + c[3])) + ((c[4] + c[5]) + (c[6] + c[7]))


def _sum32_col(p):
    r = ((p[:, 0:8] + p[:, 8:16]) + p[:, 16:24]) + p[:, 24:32]
    c = [r[:, j:j + 1] for j in range(8)]
    return ((c[0] + c[1]) + (c[2] + c[3])) + ((c[4] + c[5]) + (c[6] + c[7]))


def _dist_argmin_body(x_ref, w_ref, idx_ref, loss_ref, minv, mini, a_s, b_s):
    t = pl.program_id(0)
    k = pl.program_id(1)
    xb = x_ref[...]
    wb = w_ref[...]

    @pl.when(k == 0)
    def _():
        a_s[...] = _sum32_col(xb * xb)

    @pl.when(t == 0)
    def _():
        b_s[pl.ds(k * KB, KB)] = _sum32(wb * wb)

    a = a_s[...]
    b = b_s[pl.ds(k * KB, KB)]
    s = lax.dot_general(xb, wb, (((1,), (1,)), ((), ())))
    gio = lax.broadcasted_iota(jnp.int32, (TB, 128), 1)
    acc_v = None
    acc_i = None
    for g in range(KB // 128):
        sg = s[:, g * 128:(g + 1) * 128]
        bg = b[g * 128:(g + 1) * 128]
        dg = ((a - 2.0 * sg) + bg[None, :]) * (1.0 / D_LAT)
        if g == 0:
            acc_v = dg
            acc_i = gio + k * KB
        else:
            better = dg < acc_v
            acc_v = jnp.where(better, dg, acc_v)
            acc_i = jnp.where(better, gio + (k * KB + g * 128), acc_i)
    lmin = jnp.min(acc_v, axis=1, keepdims=True)
    lidx = jnp.min(jnp.where(acc_v == lmin, acc_i, jnp.int32(2**30)),
                   axis=1, keepdims=True)

    @pl.when(k == 0)
    def _():
        minv[...] = lmin.astype(jnp.bfloat16)
        mini[...] = lidx

    @pl.when(k > 0)
    def _():
        acc = minv[...].astype(jnp.float32)
        better = lmin < acc
        minv[...] = jnp.where(better, lmin, acc).astype(jnp.bfloat16)
        mini[...] = jnp.where(better, lidx, mini[...])

    @pl.when(k == NK - 1)
    def _():
        idx_ref[0, 0, :] = mini[...][:, 0]
        tot = jnp.sum(minv[...].astype(jnp.float32))
        acc = jnp.where(t == 0, 0.0, loss_ref[0, 0]) + tot
        loss_ref[0, 0] = acc * jnp.where(t == NT - 1,
                                         (1.0 + BETA) / N_TOK, 1.0)


_dist_argmin = pl.pallas_call(
    _dist_argmin_body,
    grid=(NT, NK),
    in_specs=[
        pl.BlockSpec((TB, D_LAT), lambda t, k: (t, 0)),
        pl.BlockSpec((KB, D_LAT), lambda t, k: (k, 0)),
    ],
    out_specs=[
        pl.BlockSpec((1, 1, TB), lambda t, k: (t, 0, 0)),
        pl.BlockSpec((1, 1), lambda t, k: (0, 0),
                     memory_space=pltpu.SMEM),
    ],
    out_shape=[
        jax.ShapeDtypeStruct((NT, 1, TB), jnp.int32),
        jax.ShapeDtypeStruct((1, 1), jnp.float32),
    ],
    scratch_shapes=[
        pltpu.VMEM((TB, 1), jnp.bfloat16),
        pltpu.VMEM((TB, 1), jnp.int32),
        pltpu.VMEM((TB, 1), jnp.float32),
        pltpu.VMEM((K_CODES,), jnp.float32),
    ],
)


@functools.cache
def _make_sc_gather():
    @functools.partial(
        pl.kernel,
        out_type=jax.ShapeDtypeStruct((N_TOK, D_LAT), jnp.float32),
        mesh=plsc.VectorSubcoreMesh(core_axis_name="c", subcore_axis_name="s",
                                    num_cores=_NC, num_subcores=_NS),
        scratch_types=[
            pltpu.VMEM((_CHUNK,), jnp.int32),
            pltpu.VMEM((_CHUNK, D_LAT), jnp.float32),
            pltpu.SemaphoreType.DMA,
        ],
        compiler_params=pltpu.CompilerParams(use_tc_tiling_on_sc=False),
    )
    def _sc_gather(idx_hbm, w_hbm, out_hbm, idx_v, rows_v, sem):
        c = lax.axis_index("c")
        s = lax.axis_index("s")
        wid = s * _NC + c
        for j in range(_CHUNKS_PER_W):
            r = wid * _CHUNKS_PER_W + j
            pltpu.sync_copy(idx_hbm.at[r], idx_v)
            pltpu.async_copy(w_hbm.at[idx_v], rows_v, sem).wait()
            pltpu.sync_copy(rows_v, out_hbm.at[pl.ds(r * _CHUNK, _CHUNK)])

    return _sc_gather


def kernel(x, W):
    xp = jnp.transpose(x, (0, 2, 3, 1))
    flat = xp.reshape(N_TOK, D_LAT)
    idx3, loss_arr = _dist_argmin(flat, W)
    idx = idx3.reshape(N_TOK)
    q = _make_sc_gather()(idx.reshape(_CHUNKS_TOTAL, _CHUNK), W)
    out = jnp.transpose(q.reshape(xp.shape), (0, 3, 1, 2))
    return (loss_arr[0, 0], idx.reshape(N_TOK, 1), out)


# speedup vs baseline: 1.7230x; 1.0478x over previous
import functools

import jax
import jax.numpy as jnp
from jax import lax
from jax.experimental import pallas as pl
from jax.experimental.pallas import tpu as pltpu
from jax.experimental.pallas import tpu_sc as plsc

K_CODES = 8192
D_LAT = 32
N_TOK = 8192
BETA = 0.25

TB = 2048
KB = 2048
NT = N_TOK // TB
NK = K_CODES // KB

_NC = 2
_NS = 16
_NW = _NC * _NS
_CHUNK = 128
_CHUNKS_TOTAL = N_TOK // _CHUNK
_CHUNKS_PER_W = _CHUNKS_TOTAL // _NW


def _sum32(p):
    r = ((p[:, 0:8] + p[:, 8:16]) + p[:, 16:24]) + p[:, 24:32]
    c = [r[:, j] for j in range(8)]
    return ((c[0] + c[1]) + (c[2] + c[3])) + ((c[4] + c[5]) + (c[6] + c[7]))


def _sum32_col(p):
    r = ((p[:, 0:8] + p[:, 8:16]) + p[:, 16:24]) + p[:, 24:32]
    c = [r[:, j:j + 1] for j in range(8)]
    return ((c[0] + c[1]) + (c[2] + c[3])) + ((c[4] + c[5]) + (c[6] + c[7]))


def _dist_argmin_body(x_ref, w_ref, idx_ref, loss_ref, minv, mini, a_s, b_s):
    t = pl.program_id(0)
    k = pl.program_id(1)
    xb = x_ref[...]
    wb = w_ref[...]

    @pl.when(k == 0)
    def _():
        a_s[...] = _sum32_col(xb * xb)

    @pl.when(t == 0)
    def _():
        b_s[pl.ds(k * KB, KB)] = _sum32(wb * wb)

    a = a_s[...]
    b = b_s[pl.ds(k * KB, KB)]
    s = lax.dot_general(xb, wb, (((1,), (1,)), ((), ())))
    gio = lax.broadcasted_iota(jnp.int32, (TB, 128), 1)
    acc_v = None
    acc_i = None
    for g in range(KB // 128):
        sg = s[:, g * 128:(g + 1) * 128]
        bg = b[g * 128:(g + 1) * 128]
        dg = ((a - 2.0 * sg) + bg[None, :]) * (1.0 / D_LAT)
        if g == 0:
            acc_v = dg
            acc_i = gio + k * KB
        else:
            better = dg < acc_v
            acc_v = jnp.where(better, dg, acc_v)
            acc_i = jnp.where(better, gio + (k * KB + g * 128), acc_i)
    lmin = jnp.min(acc_v, axis=1, keepdims=True)
    lidx = jnp.min(jnp.where(acc_v == lmin, acc_i, jnp.int32(2**30)),
                   axis=1, keepdims=True)

    @pl.when(k == 0)
    def _():
        minv[...] = lmin.astype(jnp.bfloat16)
        mini[...] = lidx

    @pl.when(k > 0)
    def _():
        acc = minv[...].astype(jnp.float32)
        better = lmin < acc
        minv[...] = jnp.where(better, lmin, acc).astype(jnp.bfloat16)
        mini[...] = jnp.where(better, lidx, mini[...])

    @pl.when(k == NK - 1)
    def _():
        idx_ref[0, 0, :] = mini[...][:, 0]
        tot = jnp.sum(minv[...].astype(jnp.float32))
        acc = jnp.where(t == 0, 0.0, loss_ref[0, 0]) + tot
        loss_ref[0, 0] = acc * jnp.where(t == NT - 1,
                                         (1.0 + BETA) / N_TOK, 1.0)


_dist_argmin = pl.pallas_call(
    _dist_argmin_body,
    grid=(NT, NK),
    in_specs=[
        pl.BlockSpec((TB, D_LAT), lambda t, k: (t, 0)),
        pl.BlockSpec((KB, D_LAT), lambda t, k: (k, 0)),
    ],
    out_specs=[
        pl.BlockSpec((1, 1, TB), lambda t, k: (t, 0, 0)),
        pl.BlockSpec((1, 1), lambda t, k: (0, 0),
                     memory_space=pltpu.SMEM),
    ],
    out_shape=[
        jax.ShapeDtypeStruct((NT, 1, TB), jnp.int32),
        jax.ShapeDtypeStruct((1, 1), jnp.float32),
    ],
    scratch_shapes=[
        pltpu.VMEM((TB, 1), jnp.bfloat16),
        pltpu.VMEM((TB, 1), jnp.int32),
        pltpu.VMEM((TB, 1), jnp.float32),
        pltpu.VMEM((K_CODES,), jnp.float32),
    ],
)


@functools.cache
def _make_sc_gather():
    @functools.partial(
        pl.kernel,
        out_type=jax.ShapeDtypeStruct((N_TOK, D_LAT), jnp.float32),
        mesh=plsc.VectorSubcoreMesh(core_axis_name="c", subcore_axis_name="s",
                                    num_cores=_NC, num_subcores=_NS),
        scratch_types=[
            pltpu.VMEM((_CHUNK,), jnp.int32),
            pltpu.VMEM((_CHUNK, D_LAT), jnp.float32),
            pltpu.SemaphoreType.DMA,
        ],
        compiler_params=pltpu.CompilerParams(use_tc_tiling_on_sc=False),
    )
    def _sc_gather(idx_hbm, w_hbm, out_hbm, idx_v, rows_v, sem):
        c = lax.axis_index("c")
        s = lax.axis_index("s")
        wid = s * _NC + c
        for j in range(_CHUNKS_PER_W):
            r = wid * _CHUNKS_PER_W + j
            pltpu.sync_copy(idx_hbm.at[r], idx_v)
            pltpu.async_copy(w_hbm.at[idx_v], rows_v, sem).wait()
            pltpu.sync_copy(rows_v, out_hbm.at[pl.ds(r * _CHUNK, _CHUNK)])

    return _sc_gather


def kernel(x, W):
    xp = jnp.transpose(x, (0, 2, 3, 1))
    flat = xp.reshape(N_TOK, D_LAT)
    idx3, loss_arr = _dist_argmin(flat, W)
    idx = idx3.reshape(N_TOK)
    q = _make_sc_gather()(idx.reshape(_CHUNKS_TOTAL, _CHUNK), W)
    out = jnp.transpose(q.reshape(xp.shape), (0, 3, 1, 2))
    return (loss_arr[0, 0], idx.reshape(N_TOK, 1), out)


# speedup vs baseline: 1.7743x; 1.0298x over previous
import functools

import jax
import jax.numpy as jnp
from jax import lax
from jax.experimental import pallas as pl
from jax.experimental.pallas import tpu as pltpu
from jax.experimental.pallas import tpu_sc as plsc

K_CODES = 8192
D_LAT = 32
N_TOK = 8192
BETA = 0.25

TB = 4096
KB = 2048
NT = N_TOK // TB
NK = K_CODES // KB

_NC = 2
_NS = 16
_NW = _NC * _NS
_CHUNK = 128
_CHUNKS_TOTAL = N_TOK // _CHUNK
_CHUNKS_PER_W = _CHUNKS_TOTAL // _NW


def _sum32(p):
    r = ((p[:, 0:8] + p[:, 8:16]) + p[:, 16:24]) + p[:, 24:32]
    c = [r[:, j] for j in range(8)]
    return ((c[0] + c[1]) + (c[2] + c[3])) + ((c[4] + c[5]) + (c[6] + c[7]))


def _sum32_col(p):
    r = ((p[:, 0:8] + p[:, 8:16]) + p[:, 16:24]) + p[:, 24:32]
    c = [r[:, j:j + 1] for j in range(8)]
    return ((c[0] + c[1]) + (c[2] + c[3])) + ((c[4] + c[5]) + (c[6] + c[7]))


def _dist_argmin_body(x_ref, w_ref, idx_ref, loss_ref, minv, mini, a_s, b_s):
    t = pl.program_id(0)
    k = pl.program_id(1)
    xb = x_ref[...]
    wb = w_ref[...]

    @pl.when(k == 0)
    def _():
        a_s[...] = _sum32_col(xb * xb)

    @pl.when(t == 0)
    def _():
        b_s[pl.ds(k * KB, KB)] = _sum32(wb * wb)

    a = a_s[...]
    b = b_s[pl.ds(k * KB, KB)]
    s = lax.dot_general(xb, wb, (((1,), (1,)), ((), ())))
    gio = lax.broadcasted_iota(jnp.int32, (TB, 128), 1)
    acc_v = None
    acc_i = None
    for g in range(KB // 128):
        sg = s[:, g * 128:(g + 1) * 128]
        bg = b[g * 128:(g + 1) * 128]
        dg = ((a - 2.0 * sg) + bg[None, :]) * (1.0 / D_LAT)
        if g == 0:
            acc_v = dg
            acc_i = gio + k * KB
        else:
            better = dg < acc_v
            acc_v = jnp.where(better, dg, acc_v)
            acc_i = jnp.where(better, gio + (k * KB + g * 128), acc_i)
    lmin = jnp.min(acc_v, axis=1, keepdims=True)
    lidx = jnp.min(jnp.where(acc_v == lmin, acc_i, jnp.int32(2**30)),
                   axis=1, keepdims=True)

    @pl.when(k == 0)
    def _():
        minv[...] = lmin.astype(jnp.bfloat16)
        mini[...] = lidx

    @pl.when(k > 0)
    def _():
        acc = minv[...].astype(jnp.float32)
        better = lmin < acc
        minv[...] = jnp.where(better, lmin, acc).astype(jnp.bfloat16)
        mini[...] = jnp.where(better, lidx, mini[...])

    @pl.when(k == NK - 1)
    def _():
        idx_ref[0, 0, :] = mini[...][:, 0]
        tot = jnp.sum(minv[...].astype(jnp.float32))
        acc = jnp.where(t == 0, 0.0, loss_ref[0, 0]) + tot
        loss_ref[0, 0] = acc * jnp.where(t == NT - 1,
                                         (1.0 + BETA) / N_TOK, 1.0)


_dist_argmin = pl.pallas_call(
    _dist_argmin_body,
    grid=(NT, NK),
    in_specs=[
        pl.BlockSpec((TB, D_LAT), lambda t, k: (t, 0)),
        pl.BlockSpec((KB, D_LAT), lambda t, k: (k, 0)),
    ],
    out_specs=[
        pl.BlockSpec((1, 1, TB), lambda t, k: (t, 0, 0)),
        pl.BlockSpec((1, 1), lambda t, k: (0, 0),
                     memory_space=pltpu.SMEM),
    ],
    out_shape=[
        jax.ShapeDtypeStruct((NT, 1, TB), jnp.int32),
        jax.ShapeDtypeStruct((1, 1), jnp.float32),
    ],
    scratch_shapes=[
        pltpu.VMEM((TB, 1), jnp.bfloat16),
        pltpu.VMEM((TB, 1), jnp.int32),
        pltpu.VMEM((TB, 1), jnp.float32),
        pltpu.VMEM((K_CODES,), jnp.float32),
    ],
)


@functools.cache
def _make_sc_gather():
    @functools.partial(
        pl.kernel,
        out_type=jax.ShapeDtypeStruct((N_TOK, D_LAT), jnp.float32),
        mesh=plsc.VectorSubcoreMesh(core_axis_name="c", subcore_axis_name="s",
                                    num_cores=_NC, num_subcores=_NS),
        scratch_types=[
            pltpu.VMEM((_CHUNK,), jnp.int32),
            pltpu.VMEM((_CHUNK, D_LAT), jnp.float32),
            pltpu.SemaphoreType.DMA,
        ],
        compiler_params=pltpu.CompilerParams(use_tc_tiling_on_sc=False),
    )
    def _sc_gather(idx_hbm, w_hbm, out_hbm, idx_v, rows_v, sem):
        c = lax.axis_index("c")
        s = lax.axis_index("s")
        wid = s * _NC + c
        for j in range(_CHUNKS_PER_W):
            r = wid * _CHUNKS_PER_W + j
            pltpu.sync_copy(idx_hbm.at[r], idx_v)
            pltpu.async_copy(w_hbm.at[idx_v], rows_v, sem).wait()
            pltpu.sync_copy(rows_v, out_hbm.at[pl.ds(r * _CHUNK, _CHUNK)])

    return _sc_gather


def kernel(x, W):
    xp = jnp.transpose(x, (0, 2, 3, 1))
    flat = xp.reshape(N_TOK, D_LAT)
    idx3, loss_arr = _dist_argmin(flat, W)
    idx = idx3.reshape(N_TOK)
    q = _make_sc_gather()(idx.reshape(_CHUNKS_TOTAL, _CHUNK), W)
    out = jnp.transpose(q.reshape(xp.shape), (0, 3, 1, 2))
    return (loss_arr[0, 0], idx.reshape(N_TOK, 1), out)
